# baseline (speedup 1.0000x reference)
import jax
import jax.numpy as jnp
from jax import lax
from jax.experimental import pallas as pl
from jax.experimental.pallas import tpu as pltpu
from jax.experimental.pallas import tpu_sc as plsc

_R = 32
_A = 6
_NW = 32
_SPT = 512
_NV = _SPT // 16


def _sc_body(x_hbm, sig_hbm, ma_hbm, mb_hbm, c1_hbm, c2_hbm, out_hbm,
             xv, euv, elv, mv,
             sufL, sufcL, sufU, sufcU,
             pcU, pU, pcL, pLv, rminv, rmaxv, outv,
             sigv, mav, mbv, m1v, m2v, ninvv,
             c1sv, c2sv, perm1v, perm2v):
    cid = lax.axis_index("c")
    sid = lax.axis_index("s")

    def _sget(ref, idx):
        return ref[pl.ds(idx, 16)][0]
    wid = sid * 2 + cid
    base = wid * _SPT

    pltpu.sync_copy(x_hbm.at[wid], xv)
    pltpu.sync_copy(sig_hbm, sigv)
    pltpu.sync_copy(ma_hbm, mav)
    pltpu.sync_copy(mb_hbm, mbv)
    pltpu.sync_copy(c1_hbm, c1sv.at[pl.ds(0, _R)])
    pltpu.sync_copy(c2_hbm, c2sv.at[pl.ds(0, _R)])

    for c in range(_R * _A // 16):
        sl = pl.ds(c * 16, 16)
        sgc = sigv[sl] + 0.0001
        ninvv[sl] = -1.0 / (2.0 * sgc * sgc)
        m1v[sl] = jnp.minimum(mav[sl], mbv[sl])
        m2v[sl] = jnp.maximum(mav[sl], mbv[sl])

    lane = lax.iota(jnp.int32, 16)

    def _perm(csv, permv):
        ch0 = csv[pl.ds(0, 16)]
        ch1 = csv[pl.ds(16, 16)]
        onei = jnp.full((16,), 1, jnp.int32)
        zeroi = jnp.zeros((16,), jnp.int32)
        lane1 = lane + 16
        r0 = zeroi
        r1 = zeroi
        for i in range(_R):
            ci = csv[pl.ds(i, 16)][0]
            tie0 = jnp.where(i < lane, onei, zeroi)
            tie1 = jnp.where(i < lane1, onei, zeroi)
            r0 = r0 + jnp.where(ci < ch0, onei,
                                jnp.where(ci == ch0, tie0, zeroi))
            r1 = r1 + jnp.where(ci < ch1, onei,
                                jnp.where(ci == ch1, tie1, zeroi))
        plsc.store_scatter(permv, [r0], lane)
        plsc.store_scatter(permv, [r1], lane1)

    _perm(c1sv, perm1v)
    _perm(c2sv, perm2v)

    minf = jnp.full((16,), -3.0e38, jnp.float32)

    @plsc.parallel_loop(0, _NV, unroll=8)
    def _init_m(i):
        mv[pl.ds(i * 16, 16)] = minf

    def _p1_r(r, _):
        m1s = [_sget(m1v, r * _A + a) for a in range(_A)]
        m2s = [_sget(m2v, r * _A + a) for a in range(_A)]
        nis = [_sget(ninvv, r * _A + a) for a in range(_A)]

        @plsc.parallel_loop(0, _NV, unroll=8)
        def _p1_i(i):
            o = i * 16
            eU = jnp.zeros((16,), jnp.float32)
            eL = jnp.zeros((16,), jnp.float32)
            for a in range(_A):
                xa = xv[pl.ds(a * _SPT + o, 16)]
                nt1 = m1s[a] - xa
                t2 = xa - m2s[a]
                du = jnp.maximum(jnp.maximum(nt1, t2), 0.0)
                dl = jnp.minimum(nt1, t2)
                eU = eU + (du * du) * nis[a]
                eL = eL + (dl * dl) * nis[a]
            euv[pl.ds((r * _NV + i) * 16, 16)] = eU
            elv[pl.ds((r * _NV + i) * 16, 16)] = eL
            mo = pl.ds(o, 16)
            mv[mo] = jnp.maximum(mv[mo], eU)
        return 0
    lax.fori_loop(0, _R, _p1_r, 0)

    def _pe_r(r, _):
        @plsc.parallel_loop(0, _NV, unroll=8)
        def _pe_i(i):
            sl = pl.ds((r * _NV + i) * 16, 16)
            mm = mv[pl.ds(i * 16, 16)]
            euv[sl] = jnp.exp(euv[sl] - mm)
            elv[sl] = jnp.exp(elv[sl] - mm)
        return 0
    lax.fori_loop(0, _R, _pe_r, 0)

    zero = jnp.zeros((16,), jnp.float32)

    @plsc.parallel_loop(0, _NV, unroll=8)
    def _z_i(i):
        sl = pl.ds((31 * _NV + i) * 16, 16)
        sufL[sl] = zero
        sufcL[sl] = zero
        sufU[sl] = zero
        sufcU[sl] = zero

    def _p2(kk, _):
        k = 30 - kk
        jl = _sget(perm1v, k + 1)
        jr = _sget(perm2v, k + 1)
        c1j = _sget(c1sv, jl)
        c2j = _sget(c2sv, jr)

        @plsc.parallel_loop(0, _NV, unroll=8)
        def _p2_i(i):
            cur = pl.ds((k * _NV + i) * 16, 16)
            nxt = pl.ds(((k + 1) * _NV + i) * 16, 16)
            Lv = elv[pl.ds((jl * _NV + i) * 16, 16)]
            sufL[cur] = sufL[nxt] + Lv
            sufcL[cur] = sufcL[nxt] + c1j * Lv
            Uv = euv[pl.ds((jr * _NV + i) * 16, 16)]
            sufU[cur] = sufU[nxt] + Uv
            sufcU[cur] = sufcU[nxt] + c2j * Uv
        return 0
    lax.fori_loop(0, _R - 1, _p2, 0)

    j0l = _sget(perm1v, 0)
    j0r = _sget(perm2v, 0)
    c1j0 = _sget(c1sv, j0l)
    c2j0 = _sget(c2sv, j0r)

    @plsc.parallel_loop(0, _NV, unroll=8)
    def _p3_init(i):
        o = pl.ds(i * 16, 16)
        s0 = pl.ds(i * 16, 16)
        Lv = elv[pl.ds((j0l * _NV + i) * 16, 16)]
        totL = sufL[s0] + Lv
        totcL = sufcL[s0] + c1j0 * Lv
        rminv[o] = totcL / totL
        Uv = euv[pl.ds((j0r * _NV + i) * 16, 16)]
        totU = sufU[s0] + Uv
        totcU = sufcU[s0] + c2j0 * Uv
        rmaxv[o] = totcU / totU
        pcU[o] = zero
        pU[o] = zero
        pcL[o] = zero
        pLv[o] = zero

    def _p3(k, _):
        jl = _sget(perm1v, k)
        jr = _sget(perm2v, k)
        c1j = _sget(c1sv, jl)
        c2j = _sget(c2sv, jr)

        @plsc.parallel_loop(0, _NV, unroll=8)
        def _p3_i(i):
            o = pl.ds(i * 16, 16)
            ks = pl.ds((k * _NV + i) * 16, 16)
            Uv = euv[pl.ds((jl * _NV + i) * 16, 16)]
            a1 = pcU[o] + c1j * Uv
            b1 = pU[o] + Uv
            pcU[o] = a1
            pU[o] = b1
            rminv[o] = jnp.minimum(rminv[o],
                                   (a1 + sufcL[ks]) / (b1 + sufL[ks]))
            Lv = elv[pl.ds((jr * _NV + i) * 16, 16)]
            e1 = pcL[o] + c2j * Lv
            f1 = pLv[o] + Lv
            pcL[o] = e1
            pLv[o] = f1
            rmaxv[o] = jnp.maximum(rmaxv[o],
                                   (e1 + sufcU[ks]) / (f1 + sufU[ks]))
        return 0
    lax.fori_loop(0, _R, _p3, 0)

    @plsc.parallel_loop(0, _NV, unroll=8)
    def _pout(i):
        o = pl.ds(i * 16, 16)
        outv[o] = (rminv[o] + rmaxv[o]) * 0.5

    pltpu.sync_copy(outv, out_hbm.at[pl.ds(base, _SPT)])


def kernel(input_data, FRB_weights, c1, c2):
    B = input_data.shape[0]
    x_pre = (input_data.T.reshape(_A, _NW, _SPT)
             .transpose(1, 0, 2).reshape(_NW, _A * _SPT))
    sig = FRB_weights[0:_R * _A]
    ma = FRB_weights[1:_R * _A + 1]
    mb = FRB_weights[2:_R * _A + 2]

    mesh = plsc.VectorSubcoreMesh(core_axis_name="c", subcore_axis_name="s")
    f = pl.kernel(
        _sc_body,
        mesh=mesh,
        out_type=jax.ShapeDtypeStruct((B,), jnp.float32),
        compiler_params=pltpu.CompilerParams(needs_layout_passes=False),
        scratch_types=[
            pltpu.VMEM((_A * _SPT,), jnp.float32),
            pltpu.VMEM((_R * _SPT,), jnp.float32),
            pltpu.VMEM((_R * _SPT,), jnp.float32),
            pltpu.VMEM((_SPT,), jnp.float32),
            pltpu.VMEM((_R * _SPT,), jnp.float32),
            pltpu.VMEM((_R * _SPT,), jnp.float32),
            pltpu.VMEM((_R * _SPT,), jnp.float32),
            pltpu.VMEM((_R * _SPT,), jnp.float32),
            pltpu.VMEM((_SPT,), jnp.float32),
            pltpu.VMEM((_SPT,), jnp.float32),
            pltpu.VMEM((_SPT,), jnp.float32),
            pltpu.VMEM((_SPT,), jnp.float32),
            pltpu.VMEM((_SPT,), jnp.float32),
            pltpu.VMEM((_SPT,), jnp.float32),
            pltpu.VMEM((_SPT,), jnp.float32),
            pltpu.VMEM((_R * _A,), jnp.float32),
            pltpu.VMEM((_R * _A,), jnp.float32),
            pltpu.VMEM((_R * _A,), jnp.float32),
            pltpu.VMEM((_R * _A + 16,), jnp.float32),
            pltpu.VMEM((_R * _A + 16,), jnp.float32),
            pltpu.VMEM((_R * _A + 16,), jnp.float32),
            pltpu.VMEM((_R + 16,), jnp.float32),
            pltpu.VMEM((_R + 16,), jnp.float32),
            pltpu.VMEM((_R + 16,), jnp.int32),
            pltpu.VMEM((_R + 16,), jnp.int32),
        ],
    )
    return f(x_pre, sig, ma, mb, c1, c2)


# speedup vs baseline: 1.1594x; 1.1594x over previous
import jax
import jax.numpy as jnp
from jax import lax
from jax.experimental import pallas as pl
from jax.experimental.pallas import tpu as pltpu
from jax.experimental.pallas import tpu_sc as plsc

_R = 32
_A = 6
_NW = 32
_SPT = 512
_NV = _SPT // 16


def _sc_body(x_hbm, sig_hbm, ma_hbm, mb_hbm, c1_hbm, c2_hbm, out_hbm,
             xv, euv, elv, mv,
             sufL, sufcL, sufU, sufcU,
             pcU, pU, pcL, pLv, rminv, rmaxv, outv,
             sigv, mav, mbv, m1v, m2v, ninvv,
             c1sv, c2sv, perm1v, perm2v):
    cid = lax.axis_index("c")
    sid = lax.axis_index("s")

    def _sget(ref, idx):
        return ref[pl.ds(idx, 16)][0]
    wid = sid * 2 + cid
    base = wid * _SPT

    pltpu.sync_copy(x_hbm.at[wid], xv)
    pltpu.sync_copy(sig_hbm, sigv)
    pltpu.sync_copy(ma_hbm, mav)
    pltpu.sync_copy(mb_hbm, mbv)
    pltpu.sync_copy(c1_hbm, c1sv.at[pl.ds(0, _R)])
    pltpu.sync_copy(c2_hbm, c2sv.at[pl.ds(0, _R)])

    for c in range(_R * _A // 16):
        sl = pl.ds(c * 16, 16)
        sgc = sigv[sl] + 0.0001
        ninvv[sl] = -1.0 / (2.0 * sgc * sgc)
        m1v[sl] = jnp.minimum(mav[sl], mbv[sl])
        m2v[sl] = jnp.maximum(mav[sl], mbv[sl])

    lane = lax.iota(jnp.int32, 16)

    def _perm(csv, permv):
        ch0 = csv[pl.ds(0, 16)]
        ch1 = csv[pl.ds(16, 16)]
        onei = jnp.full((16,), 1, jnp.int32)
        zeroi = jnp.zeros((16,), jnp.int32)
        lane1 = lane + 16
        r0 = zeroi
        r1 = zeroi
        for i in range(_R):
            ci = csv[pl.ds(i, 16)][0]
            tie0 = jnp.where(i < lane, onei, zeroi)
            tie1 = jnp.where(i < lane1, onei, zeroi)
            r0 = r0 + jnp.where(ci < ch0, onei,
                                jnp.where(ci == ch0, tie0, zeroi))
            r1 = r1 + jnp.where(ci < ch1, onei,
                                jnp.where(ci == ch1, tie1, zeroi))
        plsc.store_scatter(permv, [r0], lane)
        plsc.store_scatter(permv, [r1], lane1)

    _perm(c1sv, perm1v)
    _perm(c2sv, perm2v)

    minf = jnp.full((16,), -3.0e38, jnp.float32)

    @plsc.parallel_loop(0, _NV, unroll=2)
    def _init_m(i):
        mv[pl.ds(i * 16, 16)] = minf

    def _p1_r(r, _):
        m1s = [_sget(m1v, r * _A + a) for a in range(_A)]
        m2s = [_sget(m2v, r * _A + a) for a in range(_A)]
        nis = [_sget(ninvv, r * _A + a) for a in range(_A)]

        @plsc.parallel_loop(0, _NV, unroll=2)
        def _p1_i(i):
            o = i * 16
            eU = jnp.zeros((16,), jnp.float32)
            eL = jnp.zeros((16,), jnp.float32)
            for a in range(_A):
                xa = xv[pl.ds(a * _SPT + o, 16)]
                nt1 = m1s[a] - xa
                t2 = xa - m2s[a]
                du = jnp.maximum(jnp.maximum(nt1, t2), 0.0)
                dl = jnp.minimum(nt1, t2)
                eU = eU + (du * du) * nis[a]
                eL = eL + (dl * dl) * nis[a]
            euv[pl.ds((r * _NV + i) * 16, 16)] = eU
            elv[pl.ds((r * _NV + i) * 16, 16)] = eL
            mo = pl.ds(o, 16)
            mv[mo] = jnp.maximum(mv[mo], eU)
        return 0
    lax.fori_loop(0, _R, _p1_r, 0)

    def _pe_r(r, _):
        @plsc.parallel_loop(0, _NV, unroll=2)
        def _pe_i(i):
            sl = pl.ds((r * _NV + i) * 16, 16)
            mm = mv[pl.ds(i * 16, 16)]
            euv[sl] = jnp.exp(euv[sl] - mm)
            elv[sl] = jnp.exp(elv[sl] - mm)
        return 0
    lax.fori_loop(0, _R, _pe_r, 0)

    zero = jnp.zeros((16,), jnp.float32)

    @plsc.parallel_loop(0, _NV, unroll=2)
    def _z_i(i):
        sl = pl.ds((31 * _NV + i) * 16, 16)
        sufL[sl] = zero
        sufcL[sl] = zero
        sufU[sl] = zero
        sufcU[sl] = zero

    def _p2(kk, _):
        k = 30 - kk
        jl = _sget(perm1v, k + 1)
        jr = _sget(perm2v, k + 1)
        c1j = _sget(c1sv, jl)
        c2j = _sget(c2sv, jr)

        @plsc.parallel_loop(0, _NV, unroll=2)
        def _p2_i(i):
            cur = pl.ds((k * _NV + i) * 16, 16)
            nxt = pl.ds(((k + 1) * _NV + i) * 16, 16)
            Lv = elv[pl.ds((jl * _NV + i) * 16, 16)]
            sufL[cur] = sufL[nxt] + Lv
            sufcL[cur] = sufcL[nxt] + c1j * Lv
            Uv = euv[pl.ds((jr * _NV + i) * 16, 16)]
            sufU[cur] = sufU[nxt] + Uv
            sufcU[cur] = sufcU[nxt] + c2j * Uv
        return 0
    lax.fori_loop(0, _R - 1, _p2, 0)

    j0l = _sget(perm1v, 0)
    j0r = _sget(perm2v, 0)
    c1j0 = _sget(c1sv, j0l)
    c2j0 = _sget(c2sv, j0r)

    @plsc.parallel_loop(0, _NV, unroll=2)
    def _p3_init(i):
        o = pl.ds(i * 16, 16)
        s0 = pl.ds(i * 16, 16)
        Lv = elv[pl.ds((j0l * _NV + i) * 16, 16)]
        totL = sufL[s0] + Lv
        totcL = sufcL[s0] + c1j0 * Lv
        rminv[o] = totcL / totL
        Uv = euv[pl.ds((j0r * _NV + i) * 16, 16)]
        totU = sufU[s0] + Uv
        totcU = sufcU[s0] + c2j0 * Uv
        rmaxv[o] = totcU / totU
        pcU[o] = zero
        pU[o] = zero
        pcL[o] = zero
        pLv[o] = zero

    def _p3(k, _):
        jl = _sget(perm1v, k)
        jr = _sget(perm2v, k)
        c1j = _sget(c1sv, jl)
        c2j = _sget(c2sv, jr)

        @plsc.parallel_loop(0, _NV, unroll=2)
        def _p3_i(i):
            o = pl.ds(i * 16, 16)
            ks = pl.ds((k * _NV + i) * 16, 16)
            Uv = euv[pl.ds((jl * _NV + i) * 16, 16)]
            a1 = pcU[o] + c1j * Uv
            b1 = pU[o] + Uv
            pcU[o] = a1
            pU[o] = b1
            rminv[o] = jnp.minimum(rminv[o],
                                   (a1 + sufcL[ks]) / (b1 + sufL[ks]))
            Lv = elv[pl.ds((jr * _NV + i) * 16, 16)]
            e1 = pcL[o] + c2j * Lv
            f1 = pLv[o] + Lv
            pcL[o] = e1
            pLv[o] = f1
            rmaxv[o] = jnp.maximum(rmaxv[o],
                                   (e1 + sufcU[ks]) / (f1 + sufU[ks]))
        return 0
    lax.fori_loop(0, _R, _p3, 0)

    @plsc.parallel_loop(0, _NV, unroll=2)
    def _pout(i):
        o = pl.ds(i * 16, 16)
        outv[o] = (rminv[o] + rmaxv[o]) * 0.5

    pltpu.sync_copy(outv, out_hbm.at[pl.ds(base, _SPT)])


def kernel(input_data, FRB_weights, c1, c2):
    B = input_data.shape[0]
    x_pre = (input_data.T.reshape(_A, _NW, _SPT)
             .transpose(1, 0, 2).reshape(_NW, _A * _SPT))
    sig = FRB_weights[0:_R * _A]
    ma = FRB_weights[1:_R * _A + 1]
    mb = FRB_weights[2:_R * _A + 2]

    mesh = plsc.VectorSubcoreMesh(core_axis_name="c", subcore_axis_name="s")
    f = pl.kernel(
        _sc_body,
        mesh=mesh,
        out_type=jax.ShapeDtypeStruct((B,), jnp.float32),
        compiler_params=pltpu.CompilerParams(needs_layout_passes=False),
        scratch_types=[
            pltpu.VMEM((_A * _SPT,), jnp.float32),
            pltpu.VMEM((_R * _SPT,), jnp.float32),
            pltpu.VMEM((_R * _SPT,), jnp.float32),
            pltpu.VMEM((_SPT,), jnp.float32),
            pltpu.VMEM((_R * _SPT,), jnp.float32),
            pltpu.VMEM((_R * _SPT,), jnp.float32),
            pltpu.VMEM((_R * _SPT,), jnp.float32),
            pltpu.VMEM((_R * _SPT,), jnp.float32),
            pltpu.VMEM((_SPT,), jnp.float32),
            pltpu.VMEM((_SPT,), jnp.float32),
            pltpu.VMEM((_SPT,), jnp.float32),
            pltpu.VMEM((_SPT,), jnp.float32),
            pltpu.VMEM((_SPT,), jnp.float32),
            pltpu.VMEM((_SPT,), jnp.float32),
            pltpu.VMEM((_SPT,), jnp.float32),
            pltpu.VMEM((_R * _A,), jnp.float32),
            pltpu.VMEM((_R * _A,), jnp.float32),
            pltpu.VMEM((_R * _A,), jnp.float32),
            pltpu.VMEM((_R * _A + 16,), jnp.float32),
            pltpu.VMEM((_R * _A + 16,), jnp.float32),
            pltpu.VMEM((_R * _A + 16,), jnp.float32),
            pltpu.VMEM((_R + 16,), jnp.float32),
            pltpu.VMEM((_R + 16,), jnp.float32),
            pltpu.VMEM((_R + 16,), jnp.int32),
            pltpu.VMEM((_R + 16,), jnp.int32),
        ],
    )
    return f(x_pre, sig, ma, mb, c1, c2)


# speedup vs baseline: 1.5460x; 1.3334x over previous
import jax
import jax.numpy as jnp
from jax import lax
from jax.experimental import pallas as pl
from jax.experimental.pallas import tpu as pltpu
from jax.experimental.pallas import tpu_sc as plsc

_R = 32
_A = 6
_NW = 32

_B_SC = 4096
_SPT = _B_SC // _NW
_NV = _SPT // 16
_TC_BBLK = 6144


def _sc_body(x_hbm, sig_hbm, ma_hbm, mb_hbm, c1_hbm, c2_hbm, out_hbm,
             xv, euv, elv, mv,
             sufL, sufcL, sufU, sufcU,
             pcU, pU, pcL, pLv, rminv, rmaxv, outv,
             sigv, mav, mbv, m1v, m2v, ninvv,
             c1sv, c2sv, perm1v, perm2v):
    cid = lax.axis_index("c")
    sid = lax.axis_index("s")

    def _sget(ref, idx):
        return ref[pl.ds(idx, 16)][0]
    wid = sid * 2 + cid
    base = wid * _SPT

    pltpu.sync_copy(x_hbm.at[wid], xv)
    pltpu.sync_copy(sig_hbm, sigv)
    pltpu.sync_copy(ma_hbm, mav)
    pltpu.sync_copy(mb_hbm, mbv)
    pltpu.sync_copy(c1_hbm, c1sv.at[pl.ds(0, _R)])
    pltpu.sync_copy(c2_hbm, c2sv.at[pl.ds(0, _R)])

    for c in range(_R * _A // 16):
        sl = pl.ds(c * 16, 16)
        sgc = sigv[sl] + 0.0001
        ninvv[sl] = -1.0 / (2.0 * sgc * sgc)
        m1v[sl] = jnp.minimum(mav[sl], mbv[sl])
        m2v[sl] = jnp.maximum(mav[sl], mbv[sl])

    lane = lax.iota(jnp.int32, 16)

    def _perm(csv, permv):
        ch0 = csv[pl.ds(0, 16)]
        ch1 = csv[pl.ds(16, 16)]
        onei = jnp.full((16,), 1, jnp.int32)
        zeroi = jnp.zeros((16,), jnp.int32)
        lane1 = lane + 16
        r0 = zeroi
        r1 = zeroi
        for i in range(_R):
            ci = csv[pl.ds(i, 16)][0]
            tie0 = jnp.where(i < lane, onei, zeroi)
            tie1 = jnp.where(i < lane1, onei, zeroi)
            r0 = r0 + jnp.where(ci < ch0, onei,
                                jnp.where(ci == ch0, tie0, zeroi))
            r1 = r1 + jnp.where(ci < ch1, onei,
                                jnp.where(ci == ch1, tie1, zeroi))
        plsc.store_scatter(permv, [r0], lane)
        plsc.store_scatter(permv, [r1], lane1)

    _perm(c1sv, perm1v)
    _perm(c2sv, perm2v)

    minf = jnp.full((16,), -3.0e38, jnp.float32)

    @plsc.parallel_loop(0, _NV, unroll=4)
    def _init_m(i):
        mv[pl.ds(i * 16, 16)] = minf

    def _p1_r(r, _):
        m1s = [_sget(m1v, r * _A + a) for a in range(_A)]
        m2s = [_sget(m2v, r * _A + a) for a in range(_A)]
        nis = [_sget(ninvv, r * _A + a) for a in range(_A)]

        @plsc.parallel_loop(0, _NV, unroll=4)
        def _p1_i(i):
            o = i * 16
            eU = jnp.zeros((16,), jnp.float32)
            eL = jnp.zeros((16,), jnp.float32)
            for a in range(_A):
                xa = xv[pl.ds(a * _SPT + o, 16)]
                nt1 = m1s[a] - xa
                t2 = xa - m2s[a]
                du = jnp.maximum(jnp.maximum(nt1, t2), 0.0)
                dl = jnp.minimum(nt1, t2)
                eU = eU + (du * du) * nis[a]
                eL = eL + (dl * dl) * nis[a]
            euv[pl.ds((r * _NV + i) * 16, 16)] = eU
            elv[pl.ds((r * _NV + i) * 16, 16)] = eL
            mo = pl.ds(o, 16)
            mv[mo] = jnp.maximum(mv[mo], eU)
        return 0
    lax.fori_loop(0, _R, _p1_r, 0)

    def _pe_r(r, _):
        @plsc.parallel_loop(0, _NV, unroll=4)
        def _pe_i(i):
            sl = pl.ds((r * _NV + i) * 16, 16)
            mm = mv[pl.ds(i * 16, 16)]
            euv[sl] = jnp.exp(euv[sl] - mm)
            elv[sl] = jnp.exp(elv[sl] - mm)
        return 0
    lax.fori_loop(0, _R, _pe_r, 0)

    zero = jnp.zeros((16,), jnp.float32)

    @plsc.parallel_loop(0, _NV, unroll=4)
    def _z_i(i):
        sl = pl.ds((31 * _NV + i) * 16, 16)
        sufL[sl] = zero
        sufcL[sl] = zero
        sufU[sl] = zero
        sufcU[sl] = zero

    def _p2(kk, _):
        k = 30 - kk
        jl = _sget(perm1v, k + 1)
        jr = _sget(perm2v, k + 1)
        c1j = _sget(c1sv, jl)
        c2j = _sget(c2sv, jr)

        @plsc.parallel_loop(0, _NV, unroll=4)
        def _p2_i(i):
            cur = pl.ds((k * _NV + i) * 16, 16)
            nxt = pl.ds(((k + 1) * _NV + i) * 16, 16)
            Lv = elv[pl.ds((jl * _NV + i) * 16, 16)]
            sufL[cur] = sufL[nxt] + Lv
            sufcL[cur] = sufcL[nxt] + c1j * Lv
            Uv = euv[pl.ds((jr * _NV + i) * 16, 16)]
            sufU[cur] = sufU[nxt] + Uv
            sufcU[cur] = sufcU[nxt] + c2j * Uv
        return 0
    lax.fori_loop(0, _R - 1, _p2, 0)

    j0l = _sget(perm1v, 0)
    j0r = _sget(perm2v, 0)
    c1j0 = _sget(c1sv, j0l)
    c2j0 = _sget(c2sv, j0r)

    @plsc.parallel_loop(0, _NV, unroll=4)
    def _p3_init(i):
        o = pl.ds(i * 16, 16)
        s0 = pl.ds(i * 16, 16)
        Lv = elv[pl.ds((j0l * _NV + i) * 16, 16)]
        totL = sufL[s0] + Lv
        totcL = sufcL[s0] + c1j0 * Lv
        rminv[o] = totcL / totL
        Uv = euv[pl.ds((j0r * _NV + i) * 16, 16)]
        totU = sufU[s0] + Uv
        totcU = sufcU[s0] + c2j0 * Uv
        rmaxv[o] = totcU / totU
        pcU[o] = zero
        pU[o] = zero
        pcL[o] = zero
        pLv[o] = zero

    def _p3(k, _):
        jl = _sget(perm1v, k)
        jr = _sget(perm2v, k)
        c1j = _sget(c1sv, jl)
        c2j = _sget(c2sv, jr)

        @plsc.parallel_loop(0, _NV, unroll=4)
        def _p3_i(i):
            o = pl.ds(i * 16, 16)
            ks = pl.ds((k * _NV + i) * 16, 16)
            Uv = euv[pl.ds((jl * _NV + i) * 16, 16)]
            a1 = pcU[o] + c1j * Uv
            b1 = pU[o] + Uv
            pcU[o] = a1
            pU[o] = b1
            rminv[o] = jnp.minimum(rminv[o],
                                   (a1 + sufcL[ks]) / (b1 + sufL[ks]))
            Lv = elv[pl.ds((jr * _NV + i) * 16, 16)]
            e1 = pcL[o] + c2j * Lv
            f1 = pLv[o] + Lv
            pcL[o] = e1
            pLv[o] = f1
            rmaxv[o] = jnp.maximum(rmaxv[o],
                                   (e1 + sufcU[ks]) / (f1 + sufU[ks]))
        return 0
    lax.fori_loop(0, _R, _p3, 0)

    @plsc.parallel_loop(0, _NV, unroll=4)
    def _pout(i):
        o = pl.ds(i * 16, 16)
        outv[o] = (rminv[o] + rmaxv[o]) * 0.5

    pltpu.sync_copy(outv, out_hbm.at[pl.ds(base, _SPT)])


def _sc_call(x_sc, sig, ma, mb, c1, c2):
    B = x_sc.shape[0]
    x_pre = (x_sc.T.reshape(_A, _NW, _SPT)
             .transpose(1, 0, 2).reshape(_NW, _A * _SPT))

    mesh = plsc.VectorSubcoreMesh(core_axis_name="c", subcore_axis_name="s")
    f = pl.kernel(
        _sc_body,
        mesh=mesh,
        out_type=jax.ShapeDtypeStruct((B,), jnp.float32),
        compiler_params=pltpu.CompilerParams(needs_layout_passes=False),
        scratch_types=[
            pltpu.VMEM((_A * _SPT,), jnp.float32),
            pltpu.VMEM((_R * _SPT,), jnp.float32),
            pltpu.VMEM((_R * _SPT,), jnp.float32),
            pltpu.VMEM((_SPT,), jnp.float32),
            pltpu.VMEM((_R * _SPT,), jnp.float32),
            pltpu.VMEM((_R * _SPT,), jnp.float32),
            pltpu.VMEM((_R * _SPT,), jnp.float32),
            pltpu.VMEM((_R * _SPT,), jnp.float32),
            pltpu.VMEM((_SPT,), jnp.float32),
            pltpu.VMEM((_SPT,), jnp.float32),
            pltpu.VMEM((_SPT,), jnp.float32),
            pltpu.VMEM((_SPT,), jnp.float32),
            pltpu.VMEM((_SPT,), jnp.float32),
            pltpu.VMEM((_SPT,), jnp.float32),
            pltpu.VMEM((_SPT,), jnp.float32),
            pltpu.VMEM((_R * _A,), jnp.float32),
            pltpu.VMEM((_R * _A,), jnp.float32),
            pltpu.VMEM((_R * _A,), jnp.float32),
            pltpu.VMEM((_R * _A + 16,), jnp.float32),
            pltpu.VMEM((_R * _A + 16,), jnp.float32),
            pltpu.VMEM((_R * _A + 16,), jnp.float32),
            pltpu.VMEM((_R + 16,), jnp.float32),
            pltpu.VMEM((_R + 16,), jnp.float32),
            pltpu.VMEM((_R + 16,), jnp.int32),
            pltpu.VMEM((_R + 16,), jnp.int32),
        ],
    )
    return f(x_pre, sig, ma, mb, c1, c2)


def _rank_mask(c_col, c_row):
    ii = jax.lax.broadcasted_iota(jnp.int32, (_R, _R), 0)
    jj = jax.lax.broadcasted_iota(jnp.int32, (_R, _R), 1)
    cmp = (c_col < c_row) | ((c_col == c_row) & (ii < jj))
    rank = jnp.sum(cmp.astype(jnp.int32), axis=0, keepdims=True)
    pre = (rank <= ii).astype(jnp.float32)
    return pre, 1.0 - pre


def _tc_body(xt_ref, sig_ref, ma_ref, mb_ref,
          c1c_ref, c1r_ref, c2c_ref, c2r_ref, out_ref):
    sig = sig_ref[...] + 0.0001
    m1 = jnp.minimum(ma_ref[...], mb_ref[...])
    m2 = jnp.maximum(ma_ref[...], mb_ref[...])
    ninv = -1.0 / (2.0 * sig * sig)

    eU = jnp.zeros((_R, _TC_BBLK), jnp.float32)
    eL = jnp.zeros((_R, _TC_BBLK), jnp.float32)
    for a in range(_A):
        xa = xt_ref[a:a + 1, :]
        m1a = m1[:, a:a + 1]
        m2a = m2[:, a:a + 1]
        ninva = ninv[:, a:a + 1]
        d1 = xa - m1a
        d2 = xa - m2a
        e1 = (d1 * d1) * ninva
        e2 = (d2 * d2) * ninva
        up = jnp.where(d1 * d2 <= 0.0, 0.0, jnp.maximum(e1, e2))
        eU = eU + up
        eL = eL + jnp.minimum(e1, e2)
    emax = jnp.max(eU, axis=0, keepdims=True)
    UU = jnp.exp(eU - emax)
    LL = jnp.exp(eL - emax)

    c1c = c1c_ref[...]
    c2c = c2c_ref[...]
    M1p, M1s = _rank_mask(c1c, c1r_ref[...])
    M2p, M2s = _rank_mask(c2c, c2r_ref[...])

    def _dot(m, v):
        return jnp.dot(m, v, preferred_element_type=jnp.float32)

    c1U = c1c * UU
    c1L = c1c * LL
    s0 = jnp.sum(c1L, axis=0, keepdims=True)
    s10 = jnp.sum(LL, axis=0, keepdims=True)
    s = _dot(M1p, c1U) + _dot(M1s, c1L)
    s1 = _dot(M1p, UU) + _dot(M1s, LL)
    left = jnp.minimum(s0 / s10, jnp.min(s / s1, axis=0, keepdims=True))

    c2U = c2c * UU
    c2L = c2c * LL
    t0 = jnp.sum(c2U, axis=0, keepdims=True)
    t10 = jnp.sum(UU, axis=0, keepdims=True)
    t = _dot(M2p, c2L) + _dot(M2s, c2U)
    t1 = _dot(M2p, LL) + _dot(M2s, UU)
    right = jnp.maximum(t0 / t10, jnp.max(t / t1, axis=0, keepdims=True))

    out_ref[...] = (left + right) * 0.5


def _tc_call(x_tc, sig2, ma2, mb2, c1, c2):
    Bt = x_tc.shape[0]
    xt = x_tc.T
    c1c = c1.reshape(_R, 1)
    c1r = c1.reshape(1, _R)
    c2c = c2.reshape(_R, 1)
    c2r = c2.reshape(1, _R)
    grid = (Bt // _TC_BBLK,)
    rep = lambda i: (0, 0)
    out = pl.pallas_call(
        _tc_body,
        grid=grid,
        in_specs=[
            pl.BlockSpec((_A, _TC_BBLK), lambda i: (0, i)),
            pl.BlockSpec((_R, _A), rep),
            pl.BlockSpec((_R, _A), rep),
            pl.BlockSpec((_R, _A), rep),
            pl.BlockSpec((_R, 1), rep),
            pl.BlockSpec((1, _R), rep),
            pl.BlockSpec((_R, 1), rep),
            pl.BlockSpec((1, _R), rep),
        ],
        out_specs=pl.BlockSpec((1, _TC_BBLK), lambda i: (0, i)),
        out_shape=jax.ShapeDtypeStruct((1, Bt), jnp.float32),
        compiler_params=pltpu.CompilerParams(
            dimension_semantics=("arbitrary",),
        ),
    )(xt, sig2, ma2, mb2, c1c, c1r, c2c, c2r)
    return out.reshape(Bt)


def kernel(input_data, FRB_weights, c1, c2):
    sig = FRB_weights[0:_R * _A]
    ma = FRB_weights[1:_R * _A + 1]
    mb = FRB_weights[2:_R * _A + 2]
    x_sc = input_data[:_B_SC]
    x_tc = input_data[_B_SC:]
    out_sc = _sc_call(x_sc, sig, ma, mb, c1, c2)
    out_tc = _tc_call(x_tc, sig.reshape(_R, _A), ma.reshape(_R, _A),
                      mb.reshape(_R, _A), c1, c2)
    return jnp.concatenate([out_sc, out_tc])


# speedup vs baseline: 1.6185x; 1.0469x over previous
import jax
import jax.numpy as jnp
from jax import lax
from jax.experimental import pallas as pl
from jax.experimental.pallas import tpu as pltpu
from jax.experimental.pallas import tpu_sc as plsc

_R = 32
_A = 6
_NW = 16

_B_SC = 2048
_SPT = _B_SC // _NW
_NV = _SPT // 16
_TC_BBLK = 7168


def _sc_body(x_hbm, sig_hbm, ma_hbm, mb_hbm, c1_hbm, c2_hbm, out_hbm,
             xv, euv, elv, mv,
             sufL, sufcL, sufU, sufcU,
             pcU, pU, pcL, pLv, rminv, rmaxv, outv,
             sigv, mav, mbv, m1v, m2v, ninvv,
             c1sv, c2sv, perm1v, perm2v):
    cid = lax.axis_index("c")
    sid = lax.axis_index("s")

    def _sget(ref, idx):
        return ref[pl.ds(idx, 16)][0]
    wid = sid + cid
    base = wid * _SPT

    pltpu.sync_copy(x_hbm.at[wid], xv)
    pltpu.sync_copy(sig_hbm, sigv)
    pltpu.sync_copy(ma_hbm, mav)
    pltpu.sync_copy(mb_hbm, mbv)
    pltpu.sync_copy(c1_hbm, c1sv.at[pl.ds(0, _R)])
    pltpu.sync_copy(c2_hbm, c2sv.at[pl.ds(0, _R)])

    for c in range(_R * _A // 16):
        sl = pl.ds(c * 16, 16)
        sgc = sigv[sl] + 0.0001
        ninvv[sl] = -1.0 / (2.0 * sgc * sgc)
        m1v[sl] = jnp.minimum(mav[sl], mbv[sl])
        m2v[sl] = jnp.maximum(mav[sl], mbv[sl])

    lane = lax.iota(jnp.int32, 16)

    def _perm(csv, permv):
        ch0 = csv[pl.ds(0, 16)]
        ch1 = csv[pl.ds(16, 16)]
        onei = jnp.full((16,), 1, jnp.int32)
        zeroi = jnp.zeros((16,), jnp.int32)
        lane1 = lane + 16
        r0 = zeroi
        r1 = zeroi
        for i in range(_R):
            ci = csv[pl.ds(i, 16)][0]
            tie0 = jnp.where(i < lane, onei, zeroi)
            tie1 = jnp.where(i < lane1, onei, zeroi)
            r0 = r0 + jnp.where(ci < ch0, onei,
                                jnp.where(ci == ch0, tie0, zeroi))
            r1 = r1 + jnp.where(ci < ch1, onei,
                                jnp.where(ci == ch1, tie1, zeroi))
        plsc.store_scatter(permv, [r0], lane)
        plsc.store_scatter(permv, [r1], lane1)

    _perm(c1sv, perm1v)
    _perm(c2sv, perm2v)

    minf = jnp.full((16,), -3.0e38, jnp.float32)

    @plsc.parallel_loop(0, _NV, unroll=4)
    def _init_m(i):
        mv[pl.ds(i * 16, 16)] = minf

    def _p1_r(r, _):
        m1s = [_sget(m1v, r * _A + a) for a in range(_A)]
        m2s = [_sget(m2v, r * _A + a) for a in range(_A)]
        nis = [_sget(ninvv, r * _A + a) for a in range(_A)]

        @plsc.parallel_loop(0, _NV, unroll=4)
        def _p1_i(i):
            o = i * 16
            eU = jnp.zeros((16,), jnp.float32)
            eL = jnp.zeros((16,), jnp.float32)
            for a in range(_A):
                xa = xv[pl.ds(a * _SPT + o, 16)]
                nt1 = m1s[a] - xa
                t2 = xa - m2s[a]
                du = jnp.maximum(jnp.maximum(nt1, t2), 0.0)
                dl = jnp.minimum(nt1, t2)
                eU = eU + (du * du) * nis[a]
                eL = eL + (dl * dl) * nis[a]
            euv[pl.ds((r * _NV + i) * 16, 16)] = eU
            elv[pl.ds((r * _NV + i) * 16, 16)] = eL
            mo = pl.ds(o, 16)
            mv[mo] = jnp.maximum(mv[mo], eU)
        return 0
    lax.fori_loop(0, _R, _p1_r, 0)

    def _pe_r(r, _):
        @plsc.parallel_loop(0, _NV, unroll=4)
        def _pe_i(i):
            sl = pl.ds((r * _NV + i) * 16, 16)
            mm = mv[pl.ds(i * 16, 16)]
            euv[sl] = jnp.exp(euv[sl] - mm)
            elv[sl] = jnp.exp(elv[sl] - mm)
        return 0
    lax.fori_loop(0, _R, _pe_r, 0)

    zero = jnp.zeros((16,), jnp.float32)

    @plsc.parallel_loop(0, _NV, unroll=4)
    def _z_i(i):
        sl = pl.ds((31 * _NV + i) * 16, 16)
        sufL[sl] = zero
        sufcL[sl] = zero
        sufU[sl] = zero
        sufcU[sl] = zero

    def _p2(kk, _):
        k = 30 - kk
        jl = _sget(perm1v, k + 1)
        jr = _sget(perm2v, k + 1)
        c1j = _sget(c1sv, jl)
        c2j = _sget(c2sv, jr)

        @plsc.parallel_loop(0, _NV, unroll=4)
        def _p2_i(i):
            cur = pl.ds((k * _NV + i) * 16, 16)
            nxt = pl.ds(((k + 1) * _NV + i) * 16, 16)
            Lv = elv[pl.ds((jl * _NV + i) * 16, 16)]
            sufL[cur] = sufL[nxt] + Lv
            sufcL[cur] = sufcL[nxt] + c1j * Lv
            Uv = euv[pl.ds((jr * _NV + i) * 16, 16)]
            sufU[cur] = sufU[nxt] + Uv
            sufcU[cur] = sufcU[nxt] + c2j * Uv
        return 0
    lax.fori_loop(0, _R - 1, _p2, 0)

    j0l = _sget(perm1v, 0)
    j0r = _sget(perm2v, 0)
    c1j0 = _sget(c1sv, j0l)
    c2j0 = _sget(c2sv, j0r)

    @plsc.parallel_loop(0, _NV, unroll=4)
    def _p3_init(i):
        o = pl.ds(i * 16, 16)
        s0 = pl.ds(i * 16, 16)
        Lv = elv[pl.ds((j0l * _NV + i) * 16, 16)]
        totL = sufL[s0] + Lv
        totcL = sufcL[s0] + c1j0 * Lv
        rminv[o] = totcL / totL
        Uv = euv[pl.ds((j0r * _NV + i) * 16, 16)]
        totU = sufU[s0] + Uv
        totcU = sufcU[s0] + c2j0 * Uv
        rmaxv[o] = totcU / totU
        pcU[o] = zero
        pU[o] = zero
        pcL[o] = zero
        pLv[o] = zero

    def _p3(k, _):
        jl = _sget(perm1v, k)
        jr = _sget(perm2v, k)
        c1j = _sget(c1sv, jl)
        c2j = _sget(c2sv, jr)

        @plsc.parallel_loop(0, _NV, unroll=4)
        def _p3_i(i):
            o = pl.ds(i * 16, 16)
            ks = pl.ds((k * _NV + i) * 16, 16)
            Uv = euv[pl.ds((jl * _NV + i) * 16, 16)]
            a1 = pcU[o] + c1j * Uv
            b1 = pU[o] + Uv
            pcU[o] = a1
            pU[o] = b1
            rminv[o] = jnp.minimum(rminv[o],
                                   (a1 + sufcL[ks]) / (b1 + sufL[ks]))
            Lv = elv[pl.ds((jr * _NV + i) * 16, 16)]
            e1 = pcL[o] + c2j * Lv
            f1 = pLv[o] + Lv
            pcL[o] = e1
            pLv[o] = f1
            rmaxv[o] = jnp.maximum(rmaxv[o],
                                   (e1 + sufcU[ks]) / (f1 + sufU[ks]))
        return 0
    lax.fori_loop(0, _R, _p3, 0)

    @plsc.parallel_loop(0, _NV, unroll=4)
    def _pout(i):
        o = pl.ds(i * 16, 16)
        outv[o] = (rminv[o] + rmaxv[o]) * 0.5

    pltpu.sync_copy(outv, out_hbm.at[pl.ds(base, _SPT)])


def _sc_call(x_sc, sig, ma, mb, c1, c2):
    B = x_sc.shape[0]
    x_pre = (x_sc.T.reshape(_A, _NW, _SPT)
             .transpose(1, 0, 2).reshape(_NW, _A * _SPT))

    mesh = plsc.VectorSubcoreMesh(core_axis_name="c", subcore_axis_name="s", num_cores=1)
    f = pl.kernel(
        _sc_body,
        mesh=mesh,
        out_type=jax.ShapeDtypeStruct((B,), jnp.float32),
        compiler_params=pltpu.CompilerParams(needs_layout_passes=False),
        scratch_types=[
            pltpu.VMEM((_A * _SPT,), jnp.float32),
            pltpu.VMEM((_R * _SPT,), jnp.float32),
            pltpu.VMEM((_R * _SPT,), jnp.float32),
            pltpu.VMEM((_SPT,), jnp.float32),
            pltpu.VMEM((_R * _SPT,), jnp.float32),
            pltpu.VMEM((_R * _SPT,), jnp.float32),
            pltpu.VMEM((_R * _SPT,), jnp.float32),
            pltpu.VMEM((_R * _SPT,), jnp.float32),
            pltpu.VMEM((_SPT,), jnp.float32),
            pltpu.VMEM((_SPT,), jnp.float32),
            pltpu.VMEM((_SPT,), jnp.float32),
            pltpu.VMEM((_SPT,), jnp.float32),
            pltpu.VMEM((_SPT,), jnp.float32),
            pltpu.VMEM((_SPT,), jnp.float32),
            pltpu.VMEM((_SPT,), jnp.float32),
            pltpu.VMEM((_R * _A,), jnp.float32),
            pltpu.VMEM((_R * _A,), jnp.float32),
            pltpu.VMEM((_R * _A,), jnp.float32),
            pltpu.VMEM((_R * _A + 16,), jnp.float32),
            pltpu.VMEM((_R * _A + 16,), jnp.float32),
            pltpu.VMEM((_R * _A + 16,), jnp.float32),
            pltpu.VMEM((_R + 16,), jnp.float32),
            pltpu.VMEM((_R + 16,), jnp.float32),
            pltpu.VMEM((_R + 16,), jnp.int32),
            pltpu.VMEM((_R + 16,), jnp.int32),
        ],
    )
    return f(x_pre, sig, ma, mb, c1, c2)


def _rank_mask(c_col, c_row):
    ii = jax.lax.broadcasted_iota(jnp.int32, (_R, _R), 0)
    jj = jax.lax.broadcasted_iota(jnp.int32, (_R, _R), 1)
    cmp = (c_col < c_row) | ((c_col == c_row) & (ii < jj))
    rank = jnp.sum(cmp.astype(jnp.int32), axis=0, keepdims=True)
    pre = (rank <= ii).astype(jnp.float32)
    return pre, 1.0 - pre


def _tc_body(xt_ref, sig_ref, ma_ref, mb_ref,
          c1c_ref, c1r_ref, c2c_ref, c2r_ref, out_ref):
    sig = sig_ref[...] + 0.0001
    m1 = jnp.minimum(ma_ref[...], mb_ref[...])
    m2 = jnp.maximum(ma_ref[...], mb_ref[...])
    ninv = -1.0 / (2.0 * sig * sig)

    eU = jnp.zeros((_R, _TC_BBLK), jnp.float32)
    eL = jnp.zeros((_R, _TC_BBLK), jnp.float32)
    for a in range(_A):
        xa = xt_ref[a:a + 1, :]
        m1a = m1[:, a:a + 1]
        m2a = m2[:, a:a + 1]
        ninva = ninv[:, a:a + 1]
        d1 = xa - m1a
        d2 = xa - m2a
        e1 = (d1 * d1) * ninva
        e2 = (d2 * d2) * ninva
        up = jnp.where(d1 * d2 <= 0.0, 0.0, jnp.maximum(e1, e2))
        eU = eU + up
        eL = eL + jnp.minimum(e1, e2)
    emax = jnp.max(eU, axis=0, keepdims=True)
    UU = jnp.exp(eU - emax)
    LL = jnp.exp(eL - emax)

    c1c = c1c_ref[...]
    c2c = c2c_ref[...]
    M1p, M1s = _rank_mask(c1c, c1r_ref[...])
    M2p, M2s = _rank_mask(c2c, c2r_ref[...])

    def _dot(m, v):
        return jnp.dot(m, v, preferred_element_type=jnp.float32)

    c1U = c1c * UU
    c1L = c1c * LL
    s0 = jnp.sum(c1L, axis=0, keepdims=True)
    s10 = jnp.sum(LL, axis=0, keepdims=True)
    s = _dot(M1p, c1U) + _dot(M1s, c1L)
    s1 = _dot(M1p, UU) + _dot(M1s, LL)
    left = jnp.minimum(s0 / s10, jnp.min(s / s1, axis=0, keepdims=True))

    c2U = c2c * UU
    c2L = c2c * LL
    t0 = jnp.sum(c2U, axis=0, keepdims=True)
    t10 = jnp.sum(UU, axis=0, keepdims=True)
    t = _dot(M2p, c2L) + _dot(M2s, c2U)
    t1 = _dot(M2p, LL) + _dot(M2s, UU)
    right = jnp.maximum(t0 / t10, jnp.max(t / t1, axis=0, keepdims=True))

    out_ref[...] = (left + right) * 0.5


def _tc_call(x_tc, sig2, ma2, mb2, c1, c2):
    Bt = x_tc.shape[0]
    xt = x_tc.T
    c1c = c1.reshape(_R, 1)
    c1r = c1.reshape(1, _R)
    c2c = c2.reshape(_R, 1)
    c2r = c2.reshape(1, _R)
    grid = (Bt // _TC_BBLK,)
    rep = lambda i: (0, 0)
    out = pl.pallas_call(
        _tc_body,
        grid=grid,
        in_specs=[
            pl.BlockSpec((_A, _TC_BBLK), lambda i: (0, i)),
            pl.BlockSpec((_R, _A), rep),
            pl.BlockSpec((_R, _A), rep),
            pl.BlockSpec((_R, _A), rep),
            pl.BlockSpec((_R, 1), rep),
            pl.BlockSpec((1, _R), rep),
            pl.BlockSpec((_R, 1), rep),
            pl.BlockSpec((1, _R), rep),
        ],
        out_specs=pl.BlockSpec((1, _TC_BBLK), lambda i: (0, i)),
        out_shape=jax.ShapeDtypeStruct((1, Bt), jnp.float32),
        compiler_params=pltpu.CompilerParams(
            dimension_semantics=("arbitrary",),
        ),
    )(xt, sig2, ma2, mb2, c1c, c1r, c2c, c2r)
    return out.reshape(Bt)


def kernel(input_data, FRB_weights, c1, c2):
    sig = FRB_weights[0:_R * _A]
    ma = FRB_weights[1:_R * _A + 1]
    mb = FRB_weights[2:_R * _A + 2]
    x_sc = input_data[:_B_SC]
    x_tc = input_data[_B_SC:]
    out_sc = _sc_call(x_sc, sig, ma, mb, c1, c2)
    out_tc = _tc_call(x_tc, sig.reshape(_R, _A), ma.reshape(_R, _A),
                      mb.reshape(_R, _A), c1, c2)
    return jnp.concatenate([out_sc, out_tc])
